# Initial kernel scaffold; baseline (speedup 1.0000x reference)
import jax
import jax.numpy as jnp
from jax.experimental import pallas as pl


def kernel(adj_indices, adj_values, embedding):
    raise NotImplementedError("write your pallas kernel here")


# speedup vs baseline: 1.4602x; 1.4602x over previous
import functools

import jax
import jax.numpy as jnp
from jax import lax
from jax.experimental import pallas as pl
from jax.experimental.pallas import tpu as pltpu
from jax.experimental.pallas import tpu_sc as plsc

N = 50000
E = 800000
D = 100
DT = 128
NC = 2
NS = 16
NW = NC * NS
NSWEEP = DT // NW

B = 4096
NBLK = 196
EPAD = NBLK * B

CO = 2000
NCO = N // CO


def _make_layer(last):
    n_out = 1 if last else 2
    out_type = tuple(
        jax.ShapeDtypeStruct((DT * N,), jnp.float32) for _ in range(n_out)
    )
    scratch = [
        pltpu.VMEM((N,), jnp.float32),
        pltpu.VMEM((N,), jnp.float32),
        pltpu.VMEM((B,), jnp.int32),
        pltpu.VMEM((B,), jnp.int32),
        pltpu.VMEM((B,), jnp.float32),
        pltpu.VMEM((B,), jnp.float32),
        pltpu.VMEM((CO,), jnp.float32),
        pltpu.SemaphoreType.DMA,
        pltpu.SemaphoreType.DMA,
    ]
    mesh = plsc.VectorSubcoreMesh(
        core_axis_name="c", subcore_axis_name="s",
        num_cores=NC, num_subcores=NS,
    )

    @functools.partial(
        pl.kernel, out_type=out_type, mesh=mesh, scratch_types=scratch,
        compiler_params=pltpu.CompilerParams(needs_layout_passes=False),
    )
    def layer(ed_h, vd_h, x_h, sum_h, *rest):
        if last:
            (out_h,) = rest[:1]
        else:
            y_h, so_h = rest[:2]
        (xbuf, ybuf, ebuf0, ebuf1, vbuf0, vbuf1,
         sbuf, sem0, sem1) = rest[n_out:]

        c = lax.axis_index("c")
        s = lax.axis_index("s")
        wid = c * NS + s

        for k in range(NSWEEP):
            dim = k * NW + wid

            @pl.when(dim < D)
            def _sweep():
                xoff = dim * N

                def zero(i, _):
                    ybuf[pl.ds(i * 16, 16)] = jnp.zeros((16,), jnp.float32)
                    return 0
                lax.fori_loop(0, N // 16, zero, 0)

                pltpu.sync_copy(x_h.at[pl.ds(xoff, N)], xbuf)

                def process(ebuf, vbuf):
                    def chunk(i, _):
                        pk = ebuf[pl.ds(i * 16, 16)]
                        r16 = lax.shift_right_logical(pk, 16)
                        c16 = jnp.bitwise_and(pk, 65535)
                        v16 = vbuf[pl.ds(i * 16, 16)]
                        xv = plsc.load_gather(xbuf, [c16])
                        plsc.addupdate_scatter(ybuf, [r16], xv * v16)
                        return 0
                    lax.fori_loop(0, B // 16, chunk, 0)

                def start(b, ebuf, vbuf, sem):
                    pltpu.make_async_copy(
                        ed_h.at[pl.ds(b * B, B)], ebuf, sem).start()
                    pltpu.make_async_copy(
                        vd_h.at[pl.ds(b * B, B)], vbuf, sem).start()

                def drain(b, ebuf, vbuf, sem):
                    pltpu.make_async_copy(
                        ed_h.at[pl.ds(b * B, B)], ebuf, sem).wait()
                    pltpu.make_async_copy(
                        vd_h.at[pl.ds(b * B, B)], vbuf, sem).wait()

                start(0, ebuf0, vbuf0, sem0)

                def pair(b2, _):
                    b = 2 * b2
                    drain(b, ebuf0, vbuf0, sem0)
                    start(b + 1, ebuf1, vbuf1, sem1)
                    process(ebuf0, vbuf0)
                    drain(b + 1, ebuf1, vbuf1, sem1)
                    nxt = jnp.minimum(b + 2, NBLK - 1)
                    start(nxt, ebuf0, vbuf0, sem0)
                    process(ebuf1, vbuf1)
                    return 0
                lax.fori_loop(0, NBLK // 2, pair, 0)
                drain(0, ebuf0, vbuf0, sem0)

                def cout(j, _):
                    off2 = xoff + j * CO
                    pltpu.sync_copy(sum_h.at[pl.ds(off2, CO)], sbuf)

                    def upd(i, _):
                        a = ybuf[pl.ds(j * CO + i * 16, 16)]
                        t = sbuf[pl.ds(i * 16, 16)]
                        if last:
                            sbuf[pl.ds(i * 16, 16)] = (a + t) * 0.25
                        else:
                            sbuf[pl.ds(i * 16, 16)] = a + t
                        return 0
                    lax.fori_loop(0, CO // 16, upd, 0)

                    if last:
                        pltpu.sync_copy(sbuf, out_h.at[pl.ds(off2, CO)])
                    else:
                        pltpu.sync_copy(
                            ybuf.at[pl.ds(j * CO, CO)],
                            y_h.at[pl.ds(off2, CO)])
                        pltpu.sync_copy(sbuf, so_h.at[pl.ds(off2, CO)])
                    return 0
                lax.fori_loop(0, NCO, cout, 0)

    return layer


_layer_mid = _make_layer(last=False)
_layer_last = _make_layer(last=True)


def kernel(adj_indices, adj_values, embedding):
    rows = adj_indices[0].astype(jnp.int32)
    cols = adj_indices[1].astype(jnp.int32)
    vals = adj_values.astype(jnp.float32)

    rows = jnp.pad(rows, (0, EPAD - E))
    cols = jnp.pad(cols, (0, EPAD - E))
    vals = jnp.pad(vals, (0, EPAD - E))

    edata = jnp.bitwise_or(lax.shift_left(rows, 16), cols)
    vdata = vals

    xT = jnp.pad(embedding.astype(jnp.float32), ((0, 0), (0, DT - D)))
    xT = xT.T.reshape(-1)

    y1, s1 = _layer_mid(edata, vdata, xT, xT)
    y2, s2 = _layer_mid(edata, vdata, y1, s1)
    outT = _layer_last(edata, vdata, y2, s2)
    if isinstance(outT, (tuple, list)):
        outT = outT[0]

    return outT.reshape(DT, N).T[:, :D]


# speedup vs baseline: 1.5280x; 1.0464x over previous
import functools

import jax
import jax.numpy as jnp
from jax import lax
from jax.experimental import pallas as pl
from jax.experimental.pallas import tpu as pltpu
from jax.experimental.pallas import tpu_sc as plsc

N = 50000
E = 800000
D = 100
DT = 128
NC = 2
NS = 16
NW = NC * NS
NSWEEP = DT // NW

B = 4096
NBLK = 196
EPAD = NBLK * B

CO = 2000
NCO = N // CO


def _make_layer(last):
    n_out = 1 if last else 2
    out_type = tuple(
        jax.ShapeDtypeStruct((DT * N,), jnp.float32) for _ in range(n_out)
    )
    scratch = [
        pltpu.VMEM((N,), jnp.float32),
        pltpu.VMEM((N,), jnp.float32),
        pltpu.VMEM((B,), jnp.int32),
        pltpu.VMEM((B,), jnp.int32),
        pltpu.VMEM((B,), jnp.float32),
        pltpu.VMEM((B,), jnp.float32),
        pltpu.VMEM((CO,), jnp.float32),
        pltpu.SemaphoreType.DMA,
        pltpu.SemaphoreType.DMA,
    ]
    mesh = plsc.VectorSubcoreMesh(
        core_axis_name="c", subcore_axis_name="s",
        num_cores=NC, num_subcores=NS,
    )

    @functools.partial(
        pl.kernel, out_type=out_type, mesh=mesh, scratch_types=scratch,
        compiler_params=pltpu.CompilerParams(needs_layout_passes=False),
    )
    def layer(ed_h, vd_h, x_h, sum_h, *rest):
        if last:
            (out_h,) = rest[:1]
        else:
            y_h, so_h = rest[:2]
        (xbuf, ybuf, ebuf0, ebuf1, vbuf0, vbuf1,
         sbuf, sem0, sem1) = rest[n_out:]

        c = lax.axis_index("c")
        s = lax.axis_index("s")
        wid = c * NS + s

        for k in range(NSWEEP):
            dim = k * NW + wid

            @pl.when(dim < D)
            def _sweep():
                xoff = dim * N

                def zero(i, _):
                    ybuf[pl.ds(i * 16, 16)] = jnp.zeros((16,), jnp.float32)
                    return 0
                lax.fori_loop(0, N // 16, zero, 0, unroll=8)

                pltpu.sync_copy(x_h.at[pl.ds(xoff, N)], xbuf)

                def process(ebuf, vbuf):
                    def chunk(i, _):
                        pk = ebuf[pl.ds(i * 16, 16)]
                        r16 = lax.shift_right_logical(pk, 16)
                        c16 = jnp.bitwise_and(pk, 65535)
                        v16 = vbuf[pl.ds(i * 16, 16)]
                        xv = plsc.load_gather(xbuf, [c16])
                        plsc.addupdate_scatter(ybuf, [r16], xv * v16)
                        return 0
                    lax.fori_loop(0, B // 16, chunk, 0, unroll=8)

                def start(b, ebuf, vbuf, sem):
                    pltpu.make_async_copy(
                        ed_h.at[pl.ds(b * B, B)], ebuf, sem).start()
                    pltpu.make_async_copy(
                        vd_h.at[pl.ds(b * B, B)], vbuf, sem).start()

                def drain(b, ebuf, vbuf, sem):
                    pltpu.make_async_copy(
                        ed_h.at[pl.ds(b * B, B)], ebuf, sem).wait()
                    pltpu.make_async_copy(
                        vd_h.at[pl.ds(b * B, B)], vbuf, sem).wait()

                start(0, ebuf0, vbuf0, sem0)

                def pair(b2, _):
                    b = 2 * b2
                    drain(b, ebuf0, vbuf0, sem0)
                    start(b + 1, ebuf1, vbuf1, sem1)
                    process(ebuf0, vbuf0)
                    drain(b + 1, ebuf1, vbuf1, sem1)
                    nxt = jnp.minimum(b + 2, NBLK - 1)
                    start(nxt, ebuf0, vbuf0, sem0)
                    process(ebuf1, vbuf1)
                    return 0
                lax.fori_loop(0, NBLK // 2, pair, 0)
                drain(0, ebuf0, vbuf0, sem0)

                def cout(j, _):
                    off2 = xoff + j * CO
                    pltpu.sync_copy(sum_h.at[pl.ds(off2, CO)], sbuf)

                    def upd(i, _):
                        a = ybuf[pl.ds(j * CO + i * 16, 16)]
                        t = sbuf[pl.ds(i * 16, 16)]
                        if last:
                            sbuf[pl.ds(i * 16, 16)] = (a + t) * 0.25
                        else:
                            sbuf[pl.ds(i * 16, 16)] = a + t
                        return 0
                    lax.fori_loop(0, CO // 16, upd, 0, unroll=5)

                    if last:
                        pltpu.sync_copy(sbuf, out_h.at[pl.ds(off2, CO)])
                    else:
                        pltpu.sync_copy(
                            ybuf.at[pl.ds(j * CO, CO)],
                            y_h.at[pl.ds(off2, CO)])
                        pltpu.sync_copy(sbuf, so_h.at[pl.ds(off2, CO)])
                    return 0
                lax.fori_loop(0, NCO, cout, 0)

    return layer


_layer_mid = _make_layer(last=False)
_layer_last = _make_layer(last=True)


def kernel(adj_indices, adj_values, embedding):
    rows = adj_indices[0].astype(jnp.int32)
    cols = adj_indices[1].astype(jnp.int32)
    vals = adj_values.astype(jnp.float32)

    rows = jnp.pad(rows, (0, EPAD - E))
    cols = jnp.pad(cols, (0, EPAD - E))
    vals = jnp.pad(vals, (0, EPAD - E))

    edata = jnp.bitwise_or(lax.shift_left(rows, 16), cols)
    vdata = vals

    xT = jnp.pad(embedding.astype(jnp.float32), ((0, 0), (0, DT - D)))
    xT = xT.T.reshape(-1)

    y1, s1 = _layer_mid(edata, vdata, xT, xT)
    y2, s2 = _layer_mid(edata, vdata, y1, s1)
    outT = _layer_last(edata, vdata, y2, s2)
    if isinstance(outT, (tuple, list)):
        outT = outT[0]

    return outT.reshape(DT, N).T[:, :D]
